# Initial kernel scaffold; baseline (speedup 1.0000x reference)
import jax
import jax.numpy as jnp
from jax.experimental import pallas as pl


def kernel(points, residue_starts):
    raise NotImplementedError("write your pallas kernel here")


# speedup vs baseline: 11.0062x; 11.0062x over previous
import functools

import jax
import jax.numpy as jnp
from jax import lax
from jax.experimental import pallas as pl
from jax.experimental.pallas import tpu as pltpu
from jax.experimental.pallas import tpu_sc as plsc

B, N, D = 16, 4096, 11
R = 512
MAX_ATOMS = 24
ROW = MAX_ATOMS * D
NCHUNK = (ROW + 15) // 16
RES_PER_W = R // 2
PTS_FLAT = N * D
PTS_PAD = PTS_FLAT + 512
NB_W = RES_PER_W * ROW
NB_PAD = NB_W + 64


def _body(pts_hbm, rs_hbm, nb_hbm, cen_hbm, msk_hbm,
          pts_v, starts_v, nb_v, cen_v, msk_v):
    b = lax.axis_index("s")
    half = lax.axis_index("c")
    r0 = half * RES_PER_W

    pltpu.sync_copy(pts_hbm.at[b], pts_v.at[pl.ds(0, PTS_FLAT)])
    pltpu.sync_copy(rs_hbm.at[b], starts_v.at[pl.ds(0, R)])
    starts_v[pl.ds(R, 16)] = jnp.full((16,), N, dtype=jnp.int32)

    lane = lax.iota(jnp.int32, 16)
    zero16 = jnp.zeros((16,), jnp.float32)

    def per_residue(i, carry):
        r = r0 + i
        se = starts_v[pl.ds(r, 16)]
        s_ = se[0]
        e_ = se[1]
        cnt = e_ - s_

        nch = (cnt + 15) >> 4

        def cbody(c, acc):
            ax, ay, az = acc
            aidx = s_ + c * 16 + lane
            m = aidx < e_
            base = jnp.minimum(aidx, N - 1) * D
            gx = plsc.load_gather(pts_v, [base], mask=m)
            gy = plsc.load_gather(pts_v, [base + 1], mask=m)
            gz = plsc.load_gather(pts_v, [base + 2], mask=m)
            ax = ax + jnp.where(m, gx, 0.0)
            ay = ay + jnp.where(m, gy, 0.0)
            az = az + jnp.where(m, gz, 0.0)
            return ax, ay, az

        ax, ay, az = lax.fori_loop(0, nch, cbody, (zero16, zero16, zero16))
        cnti = cnt + lane * 0
        validv = cnti > 0
        invv = 1.0 / jnp.maximum(cnti, 1).astype(jnp.float32)
        cxv = jnp.where(validv, jnp.sum(ax) * invv, 0.0)
        cyv = jnp.where(validv, jnp.sum(ay) * invv, 0.0)
        czv = jnp.where(validv, jnp.sum(az) * invv, 0.0)

        cvec = jnp.where(lane == 0, cxv, jnp.where(lane == 1, cyv, czv))
        plsc.store_scatter(cen_v, [i * 3 + lane], cvec, mask=lane < 3)
        mvv = jnp.where(validv, 1.0, 0.0)
        plsc.store_scatter(msk_v, [i + lane * 0], mvv, mask=lane < 1)

        src0 = s_ * D
        dst0 = i * ROW
        for t in range(NCHUNK):
            flat = t * 16 + lane
            v = pts_v[pl.ds(src0 + t * 16, 16)]
            j = flat // D
            d = flat - j * D
            ct = jnp.where(d == 0, cxv, jnp.where(d == 1, cyv,
                           jnp.where(d == 2, czv, 0.0)))
            ok = (j < cnt) & (flat < ROW)
            outv = jnp.where(ok, v - ct, 0.0)
            nb_v[pl.ds(dst0 + t * 16, 16)] = outv
        return carry

    lax.fori_loop(0, RES_PER_W, per_residue, 0)

    pltpu.sync_copy(nb_v.at[pl.ds(0, NB_W)], nb_hbm.at[b, pl.ds(r0 * ROW, NB_W)])
    pltpu.sync_copy(cen_v.at[pl.ds(0, RES_PER_W * 3)],
                    cen_hbm.at[b, pl.ds(r0 * 3, RES_PER_W * 3)])
    pltpu.sync_copy(msk_v, msk_hbm.at[b, pl.ds(r0, RES_PER_W)])


@jax.jit
def _run(points_flat, residue_starts):
    mesh = plsc.VectorSubcoreMesh(core_axis_name="c", subcore_axis_name="s",
                                  num_cores=2, num_subcores=16)
    f = pl.kernel(
        _body,
        out_type=(
            jax.ShapeDtypeStruct((B, R * ROW), jnp.float32),
            jax.ShapeDtypeStruct((B, R * 3), jnp.float32),
            jax.ShapeDtypeStruct((B, R), jnp.float32),
        ),
        mesh=mesh,
        compiler_params=pltpu.CompilerParams(needs_layout_passes=False),
        scratch_types=[
            pltpu.VMEM((PTS_PAD,), jnp.float32),
            pltpu.VMEM((R + 16,), jnp.int32),
            pltpu.VMEM((NB_PAD,), jnp.float32),
            pltpu.VMEM((RES_PER_W * 3 + 16,), jnp.float32),
            pltpu.VMEM((RES_PER_W,), jnp.float32),
        ],
    )
    return f(points_flat, residue_starts)


def kernel(points, residue_starts):
    pts_flat = points.reshape(B, N * D)
    nb, cen, msk = _run(pts_flat, residue_starts)
    return (nb.reshape(B, R, MAX_ATOMS, D), cen.reshape(B, R, 3), msk)


# speedup vs baseline: 13.8953x; 1.2625x over previous
import functools

import jax
import jax.numpy as jnp
from jax import lax
from jax.experimental import pallas as pl
from jax.experimental.pallas import tpu as pltpu
from jax.experimental.pallas import tpu_sc as plsc

B, N, D = 16, 4096, 11
R = 512
MAX_ATOMS = 24
ROW = MAX_ATOMS * D
NCHUNK = (ROW + 15) // 16
RES_PER_W = R // 2
PTS_FLAT = N * D
PTS_PAD = PTS_FLAT + 512
NB_W = RES_PER_W * ROW
NB_PAD = NB_W + 64


def _body(pts_hbm, rs_hbm, nb_hbm, cen_hbm, msk_hbm,
          pts_v, starts_v, nb_v, cen_v, msk_v):
    b = lax.axis_index("s")
    half = lax.axis_index("c")
    r0 = half * RES_PER_W

    pltpu.sync_copy(pts_hbm.at[pl.ds(b * PTS_FLAT, PTS_FLAT)],
                    pts_v.at[pl.ds(0, PTS_FLAT)])
    pltpu.sync_copy(rs_hbm.at[pl.ds(b * R, R)], starts_v.at[pl.ds(0, R)])
    starts_v[pl.ds(R, 16)] = jnp.full((16,), N, dtype=jnp.int32)

    lane = lax.iota(jnp.int32, 16)
    zero16 = jnp.zeros((16,), jnp.float32)

    def per_residue(i, carry):
        r = r0 + i
        se = starts_v[pl.ds(r, 16)]
        s_ = se[0]
        e_ = se[1]
        cnt = e_ - s_

        nch = (cnt + 15) >> 4

        def cbody(c, acc):
            ax, ay, az = acc
            aidx = s_ + c * 16 + lane
            m = aidx < e_
            base = jnp.minimum(aidx, N - 1) * D
            gx = plsc.load_gather(pts_v, [base], mask=m)
            gy = plsc.load_gather(pts_v, [base + 1], mask=m)
            gz = plsc.load_gather(pts_v, [base + 2], mask=m)
            ax = ax + jnp.where(m, gx, 0.0)
            ay = ay + jnp.where(m, gy, 0.0)
            az = az + jnp.where(m, gz, 0.0)
            return ax, ay, az

        ax, ay, az = lax.fori_loop(0, nch, cbody, (zero16, zero16, zero16))
        cnti = cnt + lane * 0
        validv = cnti > 0
        invv = 1.0 / jnp.maximum(cnti, 1).astype(jnp.float32)
        cxv = jnp.where(validv, jnp.sum(ax) * invv, 0.0)
        cyv = jnp.where(validv, jnp.sum(ay) * invv, 0.0)
        czv = jnp.where(validv, jnp.sum(az) * invv, 0.0)

        cvec = jnp.where(lane == 0, cxv, jnp.where(lane == 1, cyv, czv))
        plsc.store_scatter(cen_v, [i * 3 + lane], cvec, mask=lane < 3)
        mvv = jnp.where(validv, 1.0, 0.0)
        plsc.store_scatter(msk_v, [i + lane * 0], mvv, mask=lane < 1)

        src0 = s_ * D
        dst0 = i * ROW
        for t in range(NCHUNK):
            flat = t * 16 + lane
            v = pts_v[pl.ds(src0 + t * 16, 16)]
            j = flat // D
            d = flat - j * D
            ct = jnp.where(d == 0, cxv, jnp.where(d == 1, cyv,
                           jnp.where(d == 2, czv, 0.0)))
            ok = (j < cnt) & (flat < ROW)
            outv = jnp.where(ok, v - ct, 0.0)
            nb_v[pl.ds(dst0 + t * 16, 16)] = outv
        return carry

    lax.fori_loop(0, RES_PER_W, per_residue, 0)

    pltpu.sync_copy(nb_v.at[pl.ds(0, NB_W)],
                    nb_hbm.at[pl.ds(b * (R * ROW) + r0 * ROW, NB_W)])
    pltpu.sync_copy(cen_v.at[pl.ds(0, RES_PER_W * 3)],
                    cen_hbm.at[pl.ds(b * (R * 3) + r0 * 3, RES_PER_W * 3)])
    pltpu.sync_copy(msk_v, msk_hbm.at[pl.ds(b * R + r0, RES_PER_W)])


@jax.jit
def _run(points_flat, residue_starts):
    mesh = plsc.VectorSubcoreMesh(core_axis_name="c", subcore_axis_name="s",
                                  num_cores=2, num_subcores=16)
    f = pl.kernel(
        _body,
        out_type=(
            jax.ShapeDtypeStruct((B * R * ROW,), jnp.float32),
            jax.ShapeDtypeStruct((B * R * 3,), jnp.float32),
            jax.ShapeDtypeStruct((B * R,), jnp.float32),
        ),
        mesh=mesh,
        compiler_params=pltpu.CompilerParams(needs_layout_passes=False),
        scratch_types=[
            pltpu.VMEM((PTS_PAD,), jnp.float32),
            pltpu.VMEM((R + 16,), jnp.int32),
            pltpu.VMEM((NB_PAD,), jnp.float32),
            pltpu.VMEM((RES_PER_W * 3 + 16,), jnp.float32),
            pltpu.VMEM((RES_PER_W,), jnp.float32),
        ],
    )
    return f(points_flat, residue_starts)


def kernel(points, residue_starts):
    pts_flat = points.reshape(B * N * D)
    nb, cen, msk = _run(pts_flat, residue_starts.reshape(B * R))
    return (nb.reshape(B, R, MAX_ATOMS, D), cen.reshape(B, R, 3),
            msk.reshape(B, R))


# speedup vs baseline: 15.0000x; 1.0795x over previous
import jax
import jax.numpy as jnp
from jax import lax
from jax.experimental import pallas as pl
from jax.experimental.pallas import tpu as pltpu
from jax.experimental.pallas import tpu_sc as plsc

B, N, D = 16, 4096, 11
DP = 16
R = 512
MAX_ATOMS = 24
ROWP = MAX_ATOMS * DP
RES_PER_W = R // 2
PTS_FLAT = N * DP
PTS_PAD = PTS_FLAT + ROWP
CHUNK_RES = 64
N_CHUNKS = RES_PER_W // CHUNK_RES
NB_CHUNK = CHUNK_RES * ROWP
N_GROUPS = RES_PER_W // 16


def _body(pts_hbm, rs_hbm, nb_hbm, cen_hbm, msk_hbm,
          pts_v, starts_v, nb_v, prep_v, cen_v, msk_v):
    b = lax.axis_index("s")
    half = lax.axis_index("c")
    r0 = half * RES_PER_W

    pltpu.sync_copy(pts_hbm.at[pl.ds(b * PTS_FLAT, PTS_FLAT)],
                    pts_v.at[pl.ds(0, PTS_FLAT)])
    pltpu.sync_copy(rs_hbm.at[pl.ds(b * R, R)], starts_v.at[pl.ds(0, R)])
    starts_v[pl.ds(R, 16)] = jnp.full((16,), N, dtype=jnp.int32)

    lane = lax.iota(jnp.int32, 16)
    zero16 = jnp.zeros((16,), jnp.float32)

    def group(g, carry):
        gbase = r0 + g * 16
        sv = starts_v[pl.ds(gbase, 16)]
        ev = starts_v[pl.ds(gbase + 1, 16)]
        cntv = ev - sv
        maxc = jnp.max(cntv)

        def abody(j, acc):
            ax, ay, az = acc
            idx = sv + j
            m = idx < ev
            base = jnp.minimum(idx, N - 1) * DP
            gx = plsc.load_gather(pts_v, [base], mask=m)
            gy = plsc.load_gather(pts_v, [base + 1], mask=m)
            gz = plsc.load_gather(pts_v, [base + 2], mask=m)
            ax = ax + jnp.where(m, gx, 0.0)
            ay = ay + jnp.where(m, gy, 0.0)
            az = az + jnp.where(m, gz, 0.0)
            return ax, ay, az

        ax, ay, az = lax.fori_loop(0, maxc, abody, (zero16, zero16, zero16))
        validv = cntv > 0
        invv = 1.0 / jnp.maximum(cntv, 1).astype(jnp.float32)
        cxv = jnp.where(validv, ax * invv, 0.0)
        cyv = jnp.where(validv, ay * invv, 0.0)
        czv = jnp.where(validv, az * invv, 0.0)

        msk_v[pl.ds(g * 16, 16)] = jnp.where(validv, 1.0, 0.0)
        cb = (g * 16 + lane) * 3
        plsc.store_scatter(cen_v, [cb], cxv)
        plsc.store_scatter(cen_v, [cb + 1], cyv)
        plsc.store_scatter(cen_v, [cb + 2], czv)

        pb = lane * 8 + g * 128
        plsc.store_scatter(prep_v, [pb], sv * DP)
        plsc.store_scatter(prep_v, [pb + 1], cntv)
        plsc.store_scatter(prep_v, [pb + 2], plsc.bitcast(cxv, jnp.int32))
        plsc.store_scatter(prep_v, [pb + 3], plsc.bitcast(cyv, jnp.int32))
        plsc.store_scatter(prep_v, [pb + 4], plsc.bitcast(czv, jnp.int32))
        return carry

    lax.fori_loop(0, N_GROUPS, group, 0)

    pltpu.sync_copy(cen_v.at[pl.ds(0, RES_PER_W * 3)],
                    cen_hbm.at[pl.ds(b * (R * 3) + r0 * 3, RES_PER_W * 3)])
    pltpu.sync_copy(msk_v, msk_hbm.at[pl.ds(b * R + r0, RES_PER_W)])

    is0 = lane == 0
    is1 = lane == 1
    is2 = lane == 2

    def chunk(c, carry):
        def res(k, carry2):
            i = c * CHUNK_RES + k
            pv = prep_v[pl.ds(i * 8, 16)]
            src0 = pv[0]
            cnt = pv[1]
            cf = plsc.bitcast(pv, jnp.float32)
            ctv = jnp.where(is0, cf[2], jnp.where(is1, cf[3],
                            jnp.where(is2, cf[4], 0.0)))
            dst0 = k * ROWP
            for j in range(MAX_ATOMS):
                v = pts_v[pl.ds(src0 + j * DP, 16)]
                outv = jnp.where(j < cnt, v - ctv, zero16)
                nb_v[pl.ds(dst0 + j * DP, 16)] = outv
            return carry2

        lax.fori_loop(0, CHUNK_RES, res, 0)
        pltpu.sync_copy(
            nb_v,
            nb_hbm.at[pl.ds(b * (R * ROWP) + (r0 + c * CHUNK_RES) * ROWP,
                            NB_CHUNK)])
        return carry

    lax.fori_loop(0, N_CHUNKS, chunk, 0)


@jax.jit
def _run(pts16_flat, residue_starts_flat):
    mesh = plsc.VectorSubcoreMesh(core_axis_name="c", subcore_axis_name="s",
                                  num_cores=2, num_subcores=16)
    f = pl.kernel(
        _body,
        out_type=(
            jax.ShapeDtypeStruct((B * R * ROWP,), jnp.float32),
            jax.ShapeDtypeStruct((B * R * 3,), jnp.float32),
            jax.ShapeDtypeStruct((B * R,), jnp.float32),
        ),
        mesh=mesh,
        compiler_params=pltpu.CompilerParams(needs_layout_passes=False),
        scratch_types=[
            pltpu.VMEM((PTS_PAD,), jnp.float32),
            pltpu.VMEM((R + 16,), jnp.int32),
            pltpu.VMEM((NB_CHUNK,), jnp.float32),
            pltpu.VMEM((RES_PER_W * 8,), jnp.int32),
            pltpu.VMEM((RES_PER_W * 3 + 16,), jnp.float32),
            pltpu.VMEM((RES_PER_W,), jnp.float32),
        ],
    )
    return f(pts16_flat, residue_starts_flat)


def kernel(points, residue_starts):
    pts16 = jnp.concatenate(
        [points, jnp.zeros((B, N, DP - D), jnp.float32)], axis=-1)
    nb, cen, msk = _run(pts16.reshape(B * N * DP), residue_starts.reshape(B * R))
    nb = nb.reshape(B, R, MAX_ATOMS, DP)[:, :, :, :D]
    return (nb, cen.reshape(B, R, 3), msk.reshape(B, R))


# speedup vs baseline: 62.2161x; 4.1477x over previous
import jax
import jax.numpy as jnp
from jax import lax
from jax.experimental import pallas as pl
from jax.experimental.pallas import tpu as pltpu
from jax.experimental.pallas import tpu_sc as plsc

B, N, D = 16, 4096, 11
R = 512
MAX_ATOMS = 24
RES_PER_W = R // 2
PTS_FLAT = N * D
PTS_PAD = PTS_FLAT + 16
N_GROUPS = RES_PER_W // 16
NB_ROWS = D * MAX_ATOMS


def _body(pts_hbm, rs_hbm, nb_hbm, cen_hbm, msk_hbm,
          pts_v, starts_v, nb_v, cxb, cyb, czb, msk_v):
    b = lax.axis_index("s")
    half = lax.axis_index("c")
    r0 = half * RES_PER_W

    pltpu.sync_copy(pts_hbm.at[pl.ds(b * PTS_FLAT, PTS_FLAT)],
                    pts_v.at[pl.ds(0, PTS_FLAT)])
    pltpu.sync_copy(rs_hbm.at[pl.ds(b * R, R)], starts_v.at[pl.ds(0, R)])
    starts_v[pl.ds(R, 16)] = jnp.full((16,), N, dtype=jnp.int32)

    zero16 = jnp.zeros((16,), jnp.float32)

    def group(g, carry):
        gbase = r0 + g * 16
        sv = starts_v[pl.ds(gbase, 16)]
        ev = starts_v[pl.ds(gbase + 1, 16)]
        cntv = ev - sv
        maxc = jnp.max(cntv)

        def abody(j, acc):
            ax, ay, az = acc
            idx = sv + j
            m = idx < ev
            base = jnp.minimum(idx, N - 1)
            gx = plsc.load_gather(pts_v, [base], mask=m)
            gy = plsc.load_gather(pts_v, [base + N], mask=m)
            gz = plsc.load_gather(pts_v, [base + 2 * N], mask=m)
            ax = ax + jnp.where(m, gx, 0.0)
            ay = ay + jnp.where(m, gy, 0.0)
            az = az + jnp.where(m, gz, 0.0)
            return ax, ay, az

        ax, ay, az = lax.fori_loop(0, maxc, abody, (zero16, zero16, zero16))
        validv = cntv > 0
        invv = 1.0 / jnp.maximum(cntv, 1).astype(jnp.float32)
        cxv = jnp.where(validv, ax * invv, 0.0)
        cyv = jnp.where(validv, ay * invv, 0.0)
        czv = jnp.where(validv, az * invv, 0.0)

        o = g * 16
        cxb[pl.ds(o, 16)] = cxv
        cyb[pl.ds(o, 16)] = cyv
        czb[pl.ds(o, 16)] = czv
        msk_v[pl.ds(o, 16)] = jnp.where(validv, 1.0, 0.0)
        cens = (cxv, cyv, czv)

        for j in range(MAX_ATOMS):
            mj = j < cntv
            bvj = jnp.minimum(sv + j, N - 1)
            for d in range(D):
                gv = plsc.load_gather(pts_v, [bvj + d * N], mask=mj)
                if d < 3:
                    gv = gv - cens[d]
                outv = jnp.where(mj, gv, 0.0)
                nb_v[d * MAX_ATOMS + j, pl.ds(o, 16)] = outv
        return carry

    lax.fori_loop(0, N_GROUPS, group, 0)

    pltpu.sync_copy(nb_v, nb_hbm.at[pl.ds(b * NB_ROWS, NB_ROWS),
                                    pl.ds(r0, RES_PER_W)])
    pltpu.sync_copy(cxb, cen_hbm.at[pl.ds(b * R + r0, RES_PER_W)])
    pltpu.sync_copy(cyb, cen_hbm.at[pl.ds((16 + b) * R + r0, RES_PER_W)])
    pltpu.sync_copy(czb, cen_hbm.at[pl.ds((32 + b) * R + r0, RES_PER_W)])
    pltpu.sync_copy(msk_v, msk_hbm.at[pl.ds(b * R + r0, RES_PER_W)])


@jax.jit
def _run(pts_flat, residue_starts_flat):
    mesh = plsc.VectorSubcoreMesh(core_axis_name="c", subcore_axis_name="s",
                                  num_cores=2, num_subcores=16)
    f = pl.kernel(
        _body,
        out_type=(
            jax.ShapeDtypeStruct((B * NB_ROWS, R), jnp.float32),
            jax.ShapeDtypeStruct((3 * B * R,), jnp.float32),
            jax.ShapeDtypeStruct((B * R,), jnp.float32),
        ),
        mesh=mesh,
        compiler_params=pltpu.CompilerParams(needs_layout_passes=False),
        scratch_types=[
            pltpu.VMEM((PTS_PAD,), jnp.float32),
            pltpu.VMEM((R + 16,), jnp.int32),
            pltpu.VMEM((NB_ROWS, RES_PER_W), jnp.float32),
            pltpu.VMEM((RES_PER_W,), jnp.float32),
            pltpu.VMEM((RES_PER_W,), jnp.float32),
            pltpu.VMEM((RES_PER_W,), jnp.float32),
            pltpu.VMEM((RES_PER_W,), jnp.float32),
        ],
    )
    return f(pts_flat, residue_starts_flat)


def kernel(points, residue_starts):
    pts_t = points.transpose(0, 2, 1).reshape(B * D * N)
    nb, cen, msk = _run(pts_t, residue_starts.reshape(B * R))
    nb = nb.reshape(B, D, MAX_ATOMS, R).transpose(0, 3, 2, 1)
    cen = cen.reshape(3, B, R).transpose(1, 2, 0)
    return (nb, cen, msk.reshape(B, R))
